# baseline (speedup 1.0000x reference)
import jax
import jax.numpy as jnp
from jax.experimental import pallas as pl
from jax.experimental.pallas import tpu as pltpu

_BM = 512
_NBUF = 4


def _fused_kernel(a_hbm, b_ref, w_ref, bias_ref, out_ref, h_ref, abuf, sems):
    nblk = a_hbm.shape[0] // _BM

    def _copy(blk, slot):
        pltpu.make_async_copy(
            a_hbm.at[pl.ds(blk * _BM, _BM), :],
            abuf.at[slot],
            sems.at[slot],
        ).start()

    for s in range(_NBUF):
        _copy(s, s)

    h_ref[...] = (
        jnp.dot(b_ref[...], w_ref[...], preferred_element_type=jnp.float32)
        + bias_ref[...]
    )

    def _body(i, carry):
        slot = jax.lax.rem(i, _NBUF)
        pltpu.make_async_copy(
            a_hbm.at[pl.ds(i * _BM, _BM), :],
            abuf.at[slot],
            sems.at[slot],
        ).wait()
        out_ref[pl.ds(i * _BM, _BM), :] = jnp.dot(
            abuf[slot], h_ref[...], preferred_element_type=jnp.float32
        )

        @pl.when(i + _NBUF < nblk)
        def _():
            _copy(i + _NBUF, slot)

        return carry

    jax.lax.fori_loop(0, nblk, _body, 0)


def kernel(a, b, W, bias):
    n, k = a.shape
    d_in = b.shape[1]
    d_out = W.shape[1]
    bias2d = bias.reshape(1, d_out)

    out = pl.pallas_call(
        _fused_kernel,
        in_specs=[
            pl.BlockSpec(memory_space=pltpu.HBM),
            pl.BlockSpec(memory_space=pltpu.VMEM),
            pl.BlockSpec(memory_space=pltpu.VMEM),
            pl.BlockSpec(memory_space=pltpu.VMEM),
        ],
        out_specs=pl.BlockSpec(memory_space=pltpu.VMEM),
        out_shape=jax.ShapeDtypeStruct((n, d_out), jnp.float32),
        scratch_shapes=[
            pltpu.VMEM((k, d_out), jnp.float32),
            pltpu.VMEM((_NBUF, _BM, k), jnp.float32),
            pltpu.SemaphoreType.DMA((_NBUF,)),
        ],
    )(a, b, W, bias2d)
    return out


# speedup vs baseline: 1.0039x; 1.0039x over previous
import jax
import jax.numpy as jnp
from jax.experimental import pallas as pl
from jax.experimental.pallas import tpu as pltpu

_BM = 256
_NBUF = 8


def _fused_kernel(a_hbm, b_ref, w_ref, bias_ref, out_ref, h_ref, abuf, sems):
    nblk = a_hbm.shape[0] // _BM

    def _copy(blk, slot):
        pltpu.make_async_copy(
            a_hbm.at[pl.ds(blk * _BM, _BM), :],
            abuf.at[slot],
            sems.at[slot],
        ).start()

    for s in range(_NBUF):
        _copy(s, s)

    h_ref[...] = (
        jnp.dot(b_ref[...], w_ref[...], preferred_element_type=jnp.float32)
        + bias_ref[...]
    )

    def _body(i, carry):
        slot = jax.lax.rem(i, _NBUF)
        pltpu.make_async_copy(
            a_hbm.at[pl.ds(i * _BM, _BM), :],
            abuf.at[slot],
            sems.at[slot],
        ).wait()
        out_ref[pl.ds(i * _BM, _BM), :] = jnp.dot(
            abuf[slot], h_ref[...], preferred_element_type=jnp.float32
        )

        @pl.when(i + _NBUF < nblk)
        def _():
            _copy(i + _NBUF, slot)

        return carry

    jax.lax.fori_loop(0, nblk, _body, 0)


def kernel(a, b, W, bias):
    n, k = a.shape
    d_in = b.shape[1]
    d_out = W.shape[1]
    bias2d = bias.reshape(1, d_out)

    out = pl.pallas_call(
        _fused_kernel,
        in_specs=[
            pl.BlockSpec(memory_space=pltpu.HBM),
            pl.BlockSpec(memory_space=pltpu.VMEM),
            pl.BlockSpec(memory_space=pltpu.VMEM),
            pl.BlockSpec(memory_space=pltpu.VMEM),
        ],
        out_specs=pl.BlockSpec(memory_space=pltpu.VMEM),
        out_shape=jax.ShapeDtypeStruct((n, d_out), jnp.float32),
        scratch_shapes=[
            pltpu.VMEM((k, d_out), jnp.float32),
            pltpu.VMEM((_NBUF, _BM, k), jnp.float32),
            pltpu.SemaphoreType.DMA((_NBUF,)),
        ],
    )(a, b, W, bias2d)
    return out


# speedup vs baseline: 1.1835x; 1.1790x over previous
import jax
import jax.numpy as jnp
from jax.experimental import pallas as pl
from jax.experimental.pallas import tpu as pltpu

_BM = 512


def _fused_kernel(b_ref, w_ref, bias_ref, a_ref, outt_ref, h_ref):
    @pl.when(pl.program_id(0) == 0)
    def _():
        h_ref[...] = (
            jnp.dot(b_ref[...], w_ref[...], preferred_element_type=jnp.float32)
            + bias_ref[...]
        )

    outt_ref[...] = jax.lax.dot_general(
        h_ref[...],
        a_ref[...],
        dimension_numbers=(((0,), (1,)), ((), ())),
        preferred_element_type=jnp.float32,
    )


def kernel(a, b, W, bias):
    n, k = a.shape
    d_in = b.shape[1]
    d_out = W.shape[1]
    bias2d = bias.reshape(1, d_out)

    outt = pl.pallas_call(
        _fused_kernel,
        grid=(n // _BM,),
        in_specs=[
            pl.BlockSpec((k, d_in), lambda i: (0, 0)),
            pl.BlockSpec((d_in, d_out), lambda i: (0, 0)),
            pl.BlockSpec((1, d_out), lambda i: (0, 0)),
            pl.BlockSpec((_BM, k), lambda i: (i, 0)),
        ],
        out_specs=pl.BlockSpec((d_out, _BM), lambda i: (0, i)),
        out_shape=jax.ShapeDtypeStruct((d_out, n), jnp.float32),
        scratch_shapes=[pltpu.VMEM((k, d_out), jnp.float32)],
        compiler_params=pltpu.CompilerParams(
            dimension_semantics=("arbitrary",),
        ),
    )(b, W, bias2d, a)
    return outt.T
